# baseline (speedup 1.0000x reference)
import functools

import jax
import jax.numpy as jnp
from jax import lax
from jax.experimental import pallas as pl
from jax.experimental.pallas import tpu as pltpu
from jax.experimental.pallas import tpu_sc as plsc

B = 4096
L = 200
E = 64
O = 64

NC = 2
NS = 16
NW = NC * NS
RPW = B // NW
C0 = 128
C1 = L - C0


def _make_pool():
    mesh = plsc.VectorSubcoreMesh(core_axis_name="c", subcore_axis_name="s")

    @functools.partial(
        pl.kernel,
        mesh=mesh,
        compiler_params=pltpu.CompilerParams(use_tc_tiling_on_sc=False),
        out_type=jax.ShapeDtypeStruct((B, E), jnp.float32),
        scratch_types=[
            pltpu.VMEM((RPW * L,), jnp.int32),
            pltpu.VMEM((L, E), jnp.float32),
            pltpu.VMEM((RPW, E), jnp.float32),
            pltpu.SemaphoreType.DMA,
        ],
    )
    def pool(x_hbm, table_hbm, sum_hbm, idx_v, rows_v, acc_v, sem):
        wid = lax.axis_index("s") * NC + lax.axis_index("c")
        base = wid * RPW
        pltpu.sync_copy(x_hbm.at[pl.ds(base * L, RPW * L)], idx_v)

        def row_body(r, carry):
            d0 = pltpu.async_copy(
                table_hbm.at[idx_v.at[pl.ds(r * L, C0)]],
                rows_v.at[pl.ds(0, C0)], sem)
            d1 = pltpu.async_copy(
                table_hbm.at[idx_v.at[pl.ds(r * L + C0, C1)]],
                rows_v.at[pl.ds(C0, C1)], sem)
            d0.wait()
            d1.wait()

            def red_body(j, s):
                return (s[0] + rows_v[j, pl.ds(0, 16)],
                        s[1] + rows_v[j, pl.ds(16, 16)],
                        s[2] + rows_v[j, pl.ds(32, 16)],
                        s[3] + rows_v[j, pl.ds(48, 16)])

            z = jnp.zeros((16,), jnp.float32)
            s0, s1, s2, s3 = lax.fori_loop(0, L, red_body, (z, z, z, z))
            acc_v[r, pl.ds(0, 16)] = s0
            acc_v[r, pl.ds(16, 16)] = s1
            acc_v[r, pl.ds(32, 16)] = s2
            acc_v[r, pl.ds(48, 16)] = s3
            return carry

        lax.fori_loop(0, RPW, row_body, 0)
        pltpu.sync_copy(acc_v, sum_hbm.at[pl.ds(base, RPW)])

    return pool


_pool = _make_pool()

_BM = 512


def _dense_body(s_ref, w_ref, b_ref, o_ref):
    h = jnp.dot(s_ref[...], w_ref[...], preferred_element_type=jnp.float32)
    o_ref[...] = jnp.maximum(h * (1.0 / L) + b_ref[...], 0.0)


def _dense(sums, w, b2):
    return pl.pallas_call(
        _dense_body,
        grid=(B // _BM,),
        in_specs=[
            pl.BlockSpec((_BM, E), lambda i: (i, 0)),
            pl.BlockSpec((E, O), lambda i: (0, 0)),
            pl.BlockSpec((1, O), lambda i: (0, 0)),
        ],
        out_specs=pl.BlockSpec((_BM, O), lambda i: (i, 0)),
        out_shape=jax.ShapeDtypeStruct((B, O), jnp.float32),
    )(sums, w, b2)


def kernel(x, table, W, b):
    x_flat = x.reshape(-1).astype(jnp.int32)
    sums = _pool(x_flat, table)
    return _dense(sums, W, b.reshape(1, O))


# speedup vs baseline: 1.1717x; 1.1717x over previous
import functools

import jax
import jax.numpy as jnp
from jax import lax
from jax.experimental import pallas as pl
from jax.experimental.pallas import tpu as pltpu
from jax.experimental.pallas import tpu_sc as plsc

B = 4096
L = 200
E = 64
O = 64

NC = 2
NS = 16
NW = NC * NS
RPW = B // NW
C0 = 128
C1 = L - C0


def _make_pool():
    mesh = plsc.VectorSubcoreMesh(core_axis_name="c", subcore_axis_name="s")

    @functools.partial(
        pl.kernel,
        mesh=mesh,
        compiler_params=pltpu.CompilerParams(use_tc_tiling_on_sc=False),
        out_type=jax.ShapeDtypeStruct((B, E), jnp.float32),
        scratch_types=[
            pltpu.VMEM((RPW, L), jnp.int32),
            pltpu.VMEM((L, E), jnp.float32),
            pltpu.VMEM((L, E), jnp.float32),
            pltpu.VMEM((RPW, E), jnp.float32),
            pltpu.SemaphoreType.DMA,
            pltpu.SemaphoreType.DMA,
        ],
    )
    def pool(x_hbm, table_hbm, sum_hbm, idx_v, rows0_v, rows1_v, acc_v,
             sem0, sem1):
        wid = lax.axis_index("s") * NC + lax.axis_index("c")
        base = wid * RPW
        pltpu.sync_copy(x_hbm.at[pl.ds(base, RPW)], idx_v)

        def start(r, rows_v, sem):
            pltpu.async_copy(
                table_hbm.at[idx_v.at[r, pl.ds(0, C0)]],
                rows_v.at[pl.ds(0, C0)], sem)
            pltpu.async_copy(
                table_hbm.at[idx_v.at[r, pl.ds(C0, C1)]],
                rows_v.at[pl.ds(C0, C1)], sem)

        def wait(rows_v, sem):
            pltpu.make_async_copy(
                table_hbm.at[pl.ds(0, L)], rows_v, sem).wait()

        def reduce_into(rows_v, r):
            def red_body(j, s):
                for q in range(4):
                    jj = j * 4 + q
                    s = (s[0] + rows_v[jj, pl.ds(0, 16)],
                         s[1] + rows_v[jj, pl.ds(16, 16)],
                         s[2] + rows_v[jj, pl.ds(32, 16)],
                         s[3] + rows_v[jj, pl.ds(48, 16)])
                return s

            z = jnp.zeros((16,), jnp.float32)
            s0, s1, s2, s3 = lax.fori_loop(0, L // 4, red_body, (z, z, z, z))
            acc_v[r, pl.ds(0, 16)] = s0
            acc_v[r, pl.ds(16, 16)] = s1
            acc_v[r, pl.ds(32, 16)] = s2
            acc_v[r, pl.ds(48, 16)] = s3

        start(0, rows0_v, sem0)

        def pair_body(g, carry):
            start(2 * g + 1, rows1_v, sem1)
            wait(rows0_v, sem0)
            reduce_into(rows0_v, 2 * g)

            @pl.when(g < RPW // 2 - 1)
            def _():
                start(2 * g + 2, rows0_v, sem0)

            wait(rows1_v, sem1)
            reduce_into(rows1_v, 2 * g + 1)
            return carry

        lax.fori_loop(0, RPW // 2, pair_body, 0)
        pltpu.sync_copy(acc_v, sum_hbm.at[pl.ds(base, RPW)])

    return pool


_pool = _make_pool()

_BM = 512


def _dense_body(s_ref, w_ref, b_ref, o_ref):
    h = jnp.dot(s_ref[...], w_ref[...], preferred_element_type=jnp.float32)
    o_ref[...] = jnp.maximum(h * (1.0 / L) + b_ref[...], 0.0)


def _dense(sums, w, b2):
    return pl.pallas_call(
        _dense_body,
        grid=(B // _BM,),
        in_specs=[
            pl.BlockSpec((_BM, E), lambda i: (i, 0)),
            pl.BlockSpec((E, O), lambda i: (0, 0)),
            pl.BlockSpec((1, O), lambda i: (0, 0)),
        ],
        out_specs=pl.BlockSpec((_BM, O), lambda i: (i, 0)),
        out_shape=jax.ShapeDtypeStruct((B, O), jnp.float32),
    )(sums, w, b2)


def kernel(x, table, W, b):
    sums = _pool(x.astype(jnp.int32), table)
    return _dense(sums, W, b.reshape(1, O))
